# baseline (speedup 1.0000x reference)
import functools
import math

import jax
import jax.numpy as jnp
import numpy as np
from jax import lax
from jax.experimental import pallas as pl
from jax.experimental.pallas import tpu as pltpu
from jax.experimental.pallas import tpu_sc as plsc

MAX_LEN = 512
DIM = 1024
EPS = 1e-5

L = 16
NW = 32
CHUNK = 32
ROWS = NW * MAX_LEN
HALF_ROWS = ROWS // 2
RPW = HALF_ROWS // NW
NCHUNK = RPW // CHUNK
ROWBLK = 512
NBLK_H = HALF_ROWS // ROWBLK


def _pe_table() -> jnp.ndarray:
    position = np.arange(0, MAX_LEN, dtype=np.float32)[:, None]
    div_term = np.exp(
        np.arange(0, DIM, 2, dtype=np.float32) * -(math.log(10000.0) / DIM))
    pe = np.zeros((MAX_LEN, DIM), dtype=np.float32)
    pe[:, 0::2] = np.sin(position * div_term)
    pe[:, 1::2] = np.cos(position * div_term)
    return jnp.asarray(pe)


def _sc_body(nchunk, pe_hbm, pidx_hbm, sidx_hbm, out_hbm,
             idxp_v, idxs_v, bufp_v, bufs_v, semg, semo):
    rpw = nchunk * CHUNK
    wid = lax.axis_index("s") * 2 + lax.axis_index("c")

    pltpu.sync_copy(pidx_hbm.at[wid], idxp_v)
    pltpu.sync_copy(sidx_hbm.at[wid], idxs_v)

    def start_gathers(c, h):
        pltpu.make_async_copy(
            pe_hbm.at[idxp_v.at[c]], bufp_v.at[h], semg.at[h]).start()
        pltpu.make_async_copy(
            pe_hbm.at[idxs_v.at[c]], bufs_v.at[h], semg.at[h]).start()

    def wait_gathers(c, h):
        pltpu.make_async_copy(
            pe_hbm.at[idxp_v.at[c]], bufp_v.at[h], semg.at[h]).wait()
        pltpu.make_async_copy(
            pe_hbm.at[idxs_v.at[c]], bufs_v.at[h], semg.at[h]).wait()

    def rows(c):
        return pl.ds(wid * rpw + c * CHUNK, CHUNK)

    def out_copy(c, h):
        return pltpu.make_async_copy(
            bufp_v.at[h], out_hbm.at[rows(c)], semo.at[h])

    def compute_chunk(h):
        def row_step(r, _):
            mask = jnp.int32(-65536)
            for k in range(DIM // (2 * L)):
                wa = bufp_v[h, r, pl.ds(k * L, L)]
                wb = bufs_v[h, r, pl.ds(k * L, L)]
                lo = (lax.bitcast_convert_type(
                          lax.shift_left(wa, 16), jnp.float32)
                      + lax.bitcast_convert_type(
                          lax.shift_left(wb, 16), jnp.float32))
                hi = (lax.bitcast_convert_type(
                          lax.bitwise_and(wa, mask), jnp.float32)
                      + lax.bitcast_convert_type(
                          lax.bitwise_and(wb, mask), jnp.float32))
                lw = lax.shift_right_logical(
                    lax.bitcast_convert_type(lo, jnp.int32), 16)
                hw = lax.bitwise_and(
                    lax.bitcast_convert_type(hi, jnp.int32), mask)
                bufp_v[h, r, pl.ds(k * L, L)] = lax.bitwise_or(lw, hw)
            return 0

        lax.fori_loop(0, CHUNK, row_step, 0)

    start_gathers(0, 0)

    def step(i, h):
        c = 2 * i + h

        @pl.when(c >= 1)
        def _():
            out_copy(c - 1, 1 - h).wait()

        @pl.when(c + 1 < nchunk)
        def _():
            start_gathers(c + 1, 1 - h)

        wait_gathers(c, h)
        compute_chunk(h)
        out_copy(c, h).start()

    def body(i, _):
        step(i, 0)
        step(i, 1)
        return 0

    lax.fori_loop(0, nchunk // 2, body, 0)
    out_copy(nchunk - 1, 1).wait()


def _ln_block(w, pe, gamma, beta):
    mask = jnp.int32(-65536)
    lo = lax.bitcast_convert_type(lax.shift_left(w, 16), jnp.float32)
    hi = lax.bitcast_convert_type(lax.bitwise_and(w, mask), jnp.float32)
    e = jnp.concatenate([lo, hi], axis=1) + pe
    mean = jnp.mean(e, axis=1, keepdims=True)
    cent = e - mean
    var = jnp.mean(cent * cent, axis=1, keepdims=True)
    rstd = lax.rsqrt(var + EPS)
    return cent * rstd * gamma + beta


def _tc_ln_body(w_ref, pe_ref, gamma_ref, beta_ref, out_ref):
    out_ref[...] = _ln_block(w_ref[...], pe_ref[...],
                             gamma_ref[...], beta_ref[...])


def _tc_ln_alias_body(acc_ref, w_ref, pe_ref, gamma_ref, beta_ref,
                      out_ref):
    del acc_ref
    out_ref[...] = _ln_block(w_ref[...], pe_ref[...],
                             gamma_ref[...], beta_ref[...])


SPLITS = (12, 12, 8)


def _make_gather(nb):
    nchunk = nb * MAX_LEN // NW // CHUNK
    return pl.kernel(
        functools.partial(_sc_body, nchunk),
        out_type=jax.ShapeDtypeStruct((nb * MAX_LEN, DIM // 2), jnp.int32),
        mesh=plsc.VectorSubcoreMesh(core_axis_name="c",
                                    subcore_axis_name="s"),
        scratch_types=[
            pltpu.VMEM((nchunk, CHUNK), jnp.int32),
            pltpu.VMEM((nchunk, CHUNK), jnp.int32),
            pltpu.VMEM((2, CHUNK, DIM // 2), jnp.int32),
            pltpu.VMEM((2, CHUNK, DIM // 2), jnp.int32),
            pltpu.SemaphoreType.DMA((2,)),
            pltpu.SemaphoreType.DMA((2,)),
        ],
    )


@jax.jit
def _run(pidxs, sidxs, ln_gamma, ln_beta):
    pe = _pe_table()
    pe_bf = pe.astype(jnp.bfloat16)
    pe_w = lax.bitcast_convert_type(
        jnp.stack([pe_bf[:, :DIM // 2], pe_bf[:, DIM // 2:]], axis=-1),
        jnp.int32)

    fixed_specs = [
        pl.BlockSpec((MAX_LEN, DIM), lambda i: (0, 0)),
        pl.BlockSpec((1, DIM), lambda i: (0, 0)),
        pl.BlockSpec((1, DIM), lambda i: (0, 0)),
    ]
    out_shape = jax.ShapeDtypeStruct((ROWS, DIM), jnp.float32)
    gamma2 = ln_gamma.reshape(1, DIM)
    beta2 = ln_beta.reshape(1, DIM)

    e2s = [_make_gather(nb)(pe_w, p, s)
           for nb, p, s in zip(SPLITS, pidxs, sidxs)]

    acc = None
    blk0 = 0
    for k, (nb, e2) in enumerate(zip(SPLITS, e2s)):
        nblk = nb * MAX_LEN // ROWBLK
        e2_spec = pl.BlockSpec((ROWBLK, DIM // 2), lambda i: (i, 0))
        out_spec = pl.BlockSpec((ROWBLK, DIM),
                                functools.partial(
                                    lambda off, i: (i + off, 0), blk0))
        if k == 0:
            ln = pl.pallas_call(
                _tc_ln_body,
                grid=(nblk,),
                in_specs=[e2_spec] + fixed_specs,
                out_specs=out_spec,
                out_shape=out_shape,
            )
            acc = ln(e2, pe, gamma2, beta2)
        else:
            ln = pl.pallas_call(
                _tc_ln_alias_body,
                grid=(nblk,),
                in_specs=[pl.BlockSpec(memory_space=pl.ANY), e2_spec]
                + fixed_specs,
                out_specs=out_spec,
                out_shape=out_shape,
                input_output_aliases={0: 0},
            )
            acc = ln(acc, e2, pe, gamma2, beta2)
        blk0 += nblk
    return acc


def kernel(top_vecs, sent_struct_vec, ln_gamma, ln_beta):
    B, n, _ = top_vecs.shape
    idx = sent_struct_vec.astype(jnp.int32)
    pidxs, sidxs = [], []
    b0 = 0
    for nb in SPLITS:
        nchunk = nb * MAX_LEN // NW // CHUNK
        sl = idx[b0:b0 + nb]
        pidxs.append(sl[:, :, 0].reshape(NW, nchunk, CHUNK))
        sidxs.append(sl[:, :, 1].reshape(NW, nchunk, CHUNK))
        b0 += nb
    out = _run(pidxs, sidxs, ln_gamma.astype(jnp.float32),
               ln_beta.astype(jnp.float32))
    return out.reshape(B, n, DIM)


# speedup vs baseline: 1.0533x; 1.0533x over previous
import functools
import math

import jax
import jax.numpy as jnp
import numpy as np
from jax import lax
from jax.experimental import pallas as pl
from jax.experimental.pallas import tpu as pltpu
from jax.experimental.pallas import tpu_sc as plsc

MAX_LEN = 512
DIM = 1024
EPS = 1e-5

L = 16
NW = 32
CHUNK = 32
ROWS = NW * MAX_LEN
HALF_ROWS = ROWS // 2
RPW = HALF_ROWS // NW
NCHUNK = RPW // CHUNK
ROWBLK = 512
NBLK_H = HALF_ROWS // ROWBLK


def _pe_table() -> jnp.ndarray:
    position = np.arange(0, MAX_LEN, dtype=np.float32)[:, None]
    div_term = np.exp(
        np.arange(0, DIM, 2, dtype=np.float32) * -(math.log(10000.0) / DIM))
    pe = np.zeros((MAX_LEN, DIM), dtype=np.float32)
    pe[:, 0::2] = np.sin(position * div_term)
    pe[:, 1::2] = np.cos(position * div_term)
    return jnp.asarray(pe)


def _sc_body(nchunk, pe_hbm, pidx_hbm, sidx_hbm, out_hbm,
             idxp_v, idxs_v, bufp_v, bufs_v, semg, semo):
    rpw = nchunk * CHUNK
    wid = lax.axis_index("s") * 2 + lax.axis_index("c")

    pltpu.sync_copy(pidx_hbm.at[wid], idxp_v)
    pltpu.sync_copy(sidx_hbm.at[wid], idxs_v)

    def start_gathers(c, h):
        pltpu.make_async_copy(
            pe_hbm.at[idxp_v.at[c]], bufp_v.at[h], semg.at[h]).start()
        pltpu.make_async_copy(
            pe_hbm.at[idxs_v.at[c]], bufs_v.at[h], semg.at[h]).start()

    def wait_gathers(c, h):
        pltpu.make_async_copy(
            pe_hbm.at[idxp_v.at[c]], bufp_v.at[h], semg.at[h]).wait()
        pltpu.make_async_copy(
            pe_hbm.at[idxs_v.at[c]], bufs_v.at[h], semg.at[h]).wait()

    def rows(c):
        return pl.ds(wid * rpw + c * CHUNK, CHUNK)

    def out_copy(c, h):
        return pltpu.make_async_copy(
            bufp_v.at[h], out_hbm.at[rows(c)], semo.at[h])

    def compute_chunk(h):
        def row_step(r, _):
            mask = jnp.int32(-65536)
            for k in range(DIM // (2 * L)):
                wa = bufp_v[h, r, pl.ds(k * L, L)]
                wb = bufs_v[h, r, pl.ds(k * L, L)]
                lo = (lax.bitcast_convert_type(
                          lax.shift_left(wa, 16), jnp.float32)
                      + lax.bitcast_convert_type(
                          lax.shift_left(wb, 16), jnp.float32))
                hi = (lax.bitcast_convert_type(
                          lax.bitwise_and(wa, mask), jnp.float32)
                      + lax.bitcast_convert_type(
                          lax.bitwise_and(wb, mask), jnp.float32))
                lw = lax.shift_right_logical(
                    lax.bitcast_convert_type(lo, jnp.int32), 16)
                hw = lax.bitwise_and(
                    lax.bitcast_convert_type(hi, jnp.int32), mask)
                bufp_v[h, r, pl.ds(k * L, L)] = lax.bitwise_or(lw, hw)
            return 0

        lax.fori_loop(0, CHUNK, row_step, 0)

    start_gathers(0, 0)

    def step(i, h):
        c = 2 * i + h

        @pl.when(c >= 1)
        def _():
            out_copy(c - 1, 1 - h).wait()

        @pl.when(c + 1 < nchunk)
        def _():
            start_gathers(c + 1, 1 - h)

        wait_gathers(c, h)
        compute_chunk(h)
        out_copy(c, h).start()

    def body(i, _):
        step(i, 0)
        step(i, 1)
        return 0

    lax.fori_loop(0, nchunk // 2, body, 0)
    out_copy(nchunk - 1, 1).wait()


def _ln_block(w, pe, gamma, beta):
    mask = jnp.int32(-65536)
    lo = lax.bitcast_convert_type(lax.shift_left(w, 16), jnp.float32)
    hi = lax.bitcast_convert_type(lax.bitwise_and(w, mask), jnp.float32)
    e = jnp.concatenate([lo, hi], axis=1) + pe
    mean = jnp.mean(e, axis=1, keepdims=True)
    cent = e - mean
    var = jnp.mean(cent * cent, axis=1, keepdims=True)
    rstd = lax.rsqrt(var + EPS)
    return cent * rstd * gamma + beta


def _tc_ln_body(w_ref, pe_ref, gamma_ref, beta_ref, out_ref):
    out_ref[...] = _ln_block(w_ref[...], pe_ref[...],
                             gamma_ref[...], beta_ref[...])


def _tc_ln_alias_body(acc_ref, w_ref, pe_ref, gamma_ref, beta_ref,
                      out_ref):
    del acc_ref
    out_ref[...] = _ln_block(w_ref[...], pe_ref[...],
                             gamma_ref[...], beta_ref[...])


SPLITS = (20, 12)


def _make_gather(nb):
    nchunk = nb * MAX_LEN // NW // CHUNK
    return pl.kernel(
        functools.partial(_sc_body, nchunk),
        out_type=jax.ShapeDtypeStruct((nb * MAX_LEN, DIM // 2), jnp.int32),
        mesh=plsc.VectorSubcoreMesh(core_axis_name="c",
                                    subcore_axis_name="s"),
        scratch_types=[
            pltpu.VMEM((nchunk, CHUNK), jnp.int32),
            pltpu.VMEM((nchunk, CHUNK), jnp.int32),
            pltpu.VMEM((2, CHUNK, DIM // 2), jnp.int32),
            pltpu.VMEM((2, CHUNK, DIM // 2), jnp.int32),
            pltpu.SemaphoreType.DMA((2,)),
            pltpu.SemaphoreType.DMA((2,)),
        ],
    )


@jax.jit
def _run(pidxs, sidxs, ln_gamma, ln_beta):
    pe = _pe_table()
    pe_bf = pe.astype(jnp.bfloat16)
    pe_w = lax.bitcast_convert_type(
        jnp.stack([pe_bf[:, :DIM // 2], pe_bf[:, DIM // 2:]], axis=-1),
        jnp.int32)

    fixed_specs = [
        pl.BlockSpec((MAX_LEN, DIM), lambda i: (0, 0)),
        pl.BlockSpec((1, DIM), lambda i: (0, 0)),
        pl.BlockSpec((1, DIM), lambda i: (0, 0)),
    ]
    out_shape = jax.ShapeDtypeStruct((ROWS, DIM), jnp.float32)
    gamma2 = ln_gamma.reshape(1, DIM)
    beta2 = ln_beta.reshape(1, DIM)

    e2s = [_make_gather(nb)(pe_w, p, s)
           for nb, p, s in zip(SPLITS, pidxs, sidxs)]

    acc = None
    blk0 = 0
    for k, (nb, e2) in enumerate(zip(SPLITS, e2s)):
        nblk = nb * MAX_LEN // ROWBLK
        e2_spec = pl.BlockSpec((ROWBLK, DIM // 2), lambda i: (i, 0))
        out_spec = pl.BlockSpec((ROWBLK, DIM),
                                functools.partial(
                                    lambda off, i: (i + off, 0), blk0))
        if k == 0:
            ln = pl.pallas_call(
                _tc_ln_body,
                grid=(nblk,),
                in_specs=[e2_spec] + fixed_specs,
                out_specs=out_spec,
                out_shape=out_shape,
            )
            acc = ln(e2, pe, gamma2, beta2)
        else:
            ln = pl.pallas_call(
                _tc_ln_alias_body,
                grid=(nblk,),
                in_specs=[pl.BlockSpec(memory_space=pl.ANY), e2_spec]
                + fixed_specs,
                out_specs=out_spec,
                out_shape=out_shape,
                input_output_aliases={0: 0},
            )
            acc = ln(acc, e2, pe, gamma2, beta2)
        blk0 += nblk
    return acc


def kernel(top_vecs, sent_struct_vec, ln_gamma, ln_beta):
    B, n, _ = top_vecs.shape
    idx = sent_struct_vec.astype(jnp.int32)
    pidxs, sidxs = [], []
    b0 = 0
    for nb in SPLITS:
        nchunk = nb * MAX_LEN // NW // CHUNK
        sl = idx[b0:b0 + nb]
        pidxs.append(sl[:, :, 0].reshape(NW, nchunk, CHUNK))
        sidxs.append(sl[:, :, 1].reshape(NW, nchunk, CHUNK))
        b0 += nb
    out = _run(pidxs, sidxs, ln_gamma.astype(jnp.float32),
               ln_beta.astype(jnp.float32))
    return out.reshape(B, n, DIM)
